# Initial kernel scaffold; baseline (speedup 1.0000x reference)
import jax
import jax.numpy as jnp
from jax.experimental import pallas as pl


def kernel(x):
    raise NotImplementedError("write your pallas kernel here")


# speedup vs baseline: 3.6938x; 3.6938x over previous
import dataclasses
import functools

import jax
import jax.numpy as jnp
from jax import lax
from jax.experimental import pallas as pl
from jax.experimental.pallas import tpu as pltpu
from jax.experimental.pallas import tpu_sc as plsc

_N_BITS = 4
_LEVELS = 2 ** _N_BITS

_NC = 2
_NS = 16
_NW = _NC * _NS
_L = 16

_ROWS = 1536
_COLS = 1024

_MM_BLOCK_ROWS = 16
_Q_BLOCK_ROWS = 8

_mesh = plsc.VectorSubcoreMesh(core_axis_name="c", subcore_axis_name="s")

_cparams = pltpu.CompilerParams()
if "needs_layout_passes" in pltpu.CompilerParams.__dataclass_fields__:
    _cparams = dataclasses.replace(_cparams, needs_layout_passes=False)


def _recip(x):
    bits = jax.lax.bitcast_convert_type(x, jnp.int32)
    r = jax.lax.bitcast_convert_type(jnp.int32(0x7EF311C3) - bits, jnp.float32)
    for _ in range(4):
        r = r * (2.0 - x * r)
    return r


def _reduce_block(blk_ref, vmin_ref, vmax_ref):
    nrows, ncols = blk_ref.shape

    @pl.loop(0, nrows)
    def _(r):
        @pl.loop(0, ncols, step=_L)
        def _(c):
            v = blk_ref.at[r, pl.ds(c, _L)][...]
            vmin_ref[...] = jnp.minimum(vmin_ref[...], v)
            vmax_ref[...] = jnp.maximum(vmax_ref[...], v)


@functools.partial(
    pl.kernel,
    mesh=_mesh,
    out_type=jax.ShapeDtypeStruct((2, _NW, _L), jnp.float32),
    scratch_types=[
        pltpu.VMEM((_L,), jnp.float32),
        pltpu.VMEM((_L,), jnp.float32),
    ],
)
def _minmax_kernel(x_hbm, out_hbm, vmin_ref, vmax_ref):
    vmin_ref[...] = jnp.full((_L,), jnp.inf, jnp.float32)
    vmax_ref[...] = jnp.full((_L,), -jnp.inf, jnp.float32)

    def body(x_vmem):
        _reduce_block(x_vmem, vmin_ref, vmax_ref)

    pltpu.emit_pipeline(
        body,
        grid=(_ROWS // _MM_BLOCK_ROWS,),
        in_specs=[
            pl.BlockSpec((_MM_BLOCK_ROWS, _COLS), index_map=lambda i: (i, 0))
        ],
        out_specs=[],
        core_axis_name=("c", "s"),
        dimension_semantics=(pltpu.PARALLEL,),
    )(x_hbm)

    wid = lax.axis_index("s") * _NC + lax.axis_index("c")
    pltpu.sync_copy(vmin_ref, out_hbm.at[0, wid])
    pltpu.sync_copy(vmax_ref, out_hbm.at[1, wid])


@functools.partial(
    pl.kernel,
    mesh=_mesh,
    out_type=jax.ShapeDtypeStruct((_ROWS, _COLS), jnp.float32),
    scratch_types=[pltpu.VMEM((2, _NW, _L), jnp.float32)],
    compiler_params=_cparams,
)
def _quantize_kernel(x_hbm, mm_hbm, out_hbm, mm_ref):
    pltpu.sync_copy(mm_hbm, mm_ref)

    vmin = mm_ref[0, 0]
    vmax = mm_ref[1, 0]
    for w in range(1, _NW):
        vmin = jnp.minimum(vmin, mm_ref[0, w])
        vmax = jnp.maximum(vmax, mm_ref[1, w])
    xmin = jnp.min(vmin)
    xmax = jnp.max(vmax)

    rng = xmax - xmin
    step = rng * (1.0 / _LEVELS)
    inv_step = _LEVELS * _recip(jnp.where(rng > 0, rng, 1.0))
    base = xmin + step * 0.5

    def body(x_vmem, o_vmem):
        nrows, ncols = x_vmem.shape

        @pl.loop(0, nrows)
        def _(r):
            @pl.loop(0, ncols, step=_L)
            def _(c):
                v = x_vmem.at[r, pl.ds(c, _L)][...]
                idx = ((v - xmin) * inv_step).astype(jnp.int32)
                idx = jnp.minimum(idx, _LEVELS - 1)
                q = base + idx.astype(jnp.float32) * step
                o_vmem.at[r, pl.ds(c, _L)][...] = q

    pltpu.emit_pipeline(
        body,
        grid=(_ROWS // _Q_BLOCK_ROWS,),
        in_specs=[
            pl.BlockSpec((_Q_BLOCK_ROWS, _COLS), index_map=lambda i: (i, 0))
        ],
        out_specs=[
            pl.BlockSpec((_Q_BLOCK_ROWS, _COLS), index_map=lambda i: (i, 0))
        ],
        core_axis_name=("c", "s"),
        dimension_semantics=(pltpu.PARALLEL,),
    )(x_hbm, out_hbm)


def kernel(x):
    x2d = x.reshape(_ROWS, _COLS)
    partials = _minmax_kernel(x2d)
    out2d = _quantize_kernel(x2d, partials)
    return out2d.reshape(x.shape)


# speedup vs baseline: 5.8156x; 1.5745x over previous
import dataclasses
import functools

import jax
import jax.numpy as jnp
from jax import lax
from jax.experimental import pallas as pl
from jax.experimental.pallas import tpu as pltpu
from jax.experimental.pallas import tpu_sc as plsc

_N_BITS = 4
_LEVELS = 2 ** _N_BITS

_NC = 2
_NS = 16
_L = 16

_ROWS = 1536
_COLS = 1024
_SLC = _COLS // _L

_RPT = _ROWS // _NS
_HALF = _RPT // 2
_CH = 16

_mesh = plsc.VectorSubcoreMesh(core_axis_name="c", subcore_axis_name="s")

_cparams = pltpu.CompilerParams()
if "needs_layout_passes" in pltpu.CompilerParams.__dataclass_fields__:
    _cparams = dataclasses.replace(_cparams, needs_layout_passes=False)


def _recip(x):
    bits = jax.lax.bitcast_convert_type(x, jnp.int32)
    r = jax.lax.bitcast_convert_type(jnp.int32(0x7EF311C3) - bits, jnp.float32)
    for _ in range(4):
        r = r * (2.0 - x * r)
    return r


def _tree(op, vals):
    while len(vals) > 1:
        vals = [op(vals[i], vals[i + 1]) for i in range(0, len(vals) - 1, 2)] + (
            [vals[-1]] if len(vals) % 2 else [])
    return vals[0]


@functools.partial(
    pl.kernel,
    mesh=_mesh,
    out_type=jax.ShapeDtypeStruct((_ROWS, _COLS), jnp.float32),
    scratch_types=[
        pltpu.VMEM((_RPT, _COLS), jnp.float32),
        pltpu.VMEM((2, _L), jnp.float32),
        pltpu.VMEM((_NS, 2, _L), jnp.float32),
        pltpu.VMEM_SHARED((_NS, 2, _L), jnp.float32),
        pltpu.SemaphoreType.DMA,
        pltpu.SemaphoreType.DMA,
        pltpu.SemaphoreType.DMA,
    ],
    compiler_params=_cparams,
)
def _sc_kernel(x_hbm, out_hbm, data, mm_loc, gath, shared, sem_a, sem_b, sem_o):
    cid = lax.axis_index("c")
    sid = lax.axis_index("s")
    row0 = sid * _RPT

    cp_a = pltpu.async_copy(
        x_hbm.at[pl.ds(row0, _HALF)], data.at[pl.ds(0, _HALF)], sem_a)
    cp_b = pltpu.async_copy(
        x_hbm.at[pl.ds(row0 + _HALF, _HALF)], data.at[pl.ds(_HALF, _HALF)], sem_b)

    def reduce_rows(lo, mn0, mx0):
        @pl.loop(0, _HALF, init_carry=(mn0, mx0))
        def body(r, carry):
            mn, mx = carry
            vs = [data.at[lo + r, pl.ds(c * _L, _L)][...] for c in range(_SLC)]
            mn = jnp.minimum(mn, _tree(jnp.minimum, vs))
            mx = jnp.maximum(mx, _tree(jnp.maximum, vs))
            return mn, mx
        return body

    inf = jnp.full((_L,), jnp.inf, jnp.float32)
    cp_a.wait()
    mn, mx = reduce_rows(0, inf, -inf)
    cp_b.wait()
    mn, mx = reduce_rows(_HALF, mn, mx)

    mm_loc.at[0][...] = mn
    mm_loc.at[1][...] = mx
    pltpu.sync_copy(mm_loc, shared.at[sid])
    plsc.subcore_barrier()
    pltpu.sync_copy(shared, gath)
    xmin = jnp.min(_tree(jnp.minimum, [gath[w, 0] for w in range(_NS)]))
    xmax = jnp.max(_tree(jnp.maximum, [gath[w, 1] for w in range(_NS)]))

    rng = xmax - xmin
    step = rng * (1.0 / _LEVELS)
    inv_step = _LEVELS * _recip(jnp.where(rng > 0, rng, 1.0))
    base = xmin + step * 0.5

    r0 = cid * _HALF
    copies = []
    for chunk in range(_HALF // _CH):
        lo = chunk * _CH

        @pl.loop(0, _CH)
        def _(r):
            row = r0 + lo + r
            for c in range(_SLC):
                v = data.at[row, pl.ds(c * _L, _L)][...]
                idx = ((v - xmin) * inv_step).astype(jnp.int32)
                idx = jnp.minimum(idx, _LEVELS - 1)
                q = base + idx.astype(jnp.float32) * step
                data.at[row, pl.ds(c * _L, _L)][...] = q

        copies.append(pltpu.async_copy(
            data.at[pl.ds(r0 + lo, _CH)],
            out_hbm.at[pl.ds(row0 + r0 + lo, _CH)], sem_o))
    for cp in copies:
        cp.wait()


def kernel(x):
    out2d = _sc_kernel(x.reshape(_ROWS, _COLS))
    return out2d.reshape(x.shape)


# speedup vs baseline: 13.5010x; 2.3215x over previous
import dataclasses
import functools

import jax
import jax.numpy as jnp
from jax import lax
from jax.experimental import pallas as pl
from jax.experimental.pallas import tpu as pltpu
from jax.experimental.pallas import tpu_sc as plsc

_N_BITS = 4
_LEVELS = 2 ** _N_BITS

_NC = 2
_NS = 16
_L = 16

_ROWS = 8192
_COLS = 192
_SLC = _COLS // _L

_RPT = _ROWS // _NS
_HALF = _RPT // 2
_SB = 64
_NSB = _HALF // _SB
_CH = 64

_mesh = plsc.VectorSubcoreMesh(core_axis_name="c", subcore_axis_name="s")

_cparams = pltpu.CompilerParams()
if "needs_layout_passes" in pltpu.CompilerParams.__dataclass_fields__:
    _cparams = dataclasses.replace(_cparams, needs_layout_passes=False)


def _recip(x):
    bits = jax.lax.bitcast_convert_type(x, jnp.int32)
    r = jax.lax.bitcast_convert_type(jnp.int32(0x7EF311C3) - bits, jnp.float32)
    for _ in range(4):
        r = r * (2.0 - x * r)
    return r


def _tree(op, vals):
    while len(vals) > 1:
        vals = [op(vals[i], vals[i + 1]) for i in range(0, len(vals) - 1, 2)] + (
            [vals[-1]] if len(vals) % 2 else [])
    return vals[0]


def _reduce_rows(buf, nrows, mn0, mx0):
    @pl.loop(0, nrows, init_carry=(mn0, mx0), unroll=2)
    def result(r, carry):
        mn, mx = carry
        vs = [buf.at[r, pl.ds(c * _L, _L)][...] for c in range(_SLC)]
        mn = jnp.minimum(mn, _tree(jnp.minimum, vs))
        mx = jnp.maximum(mx, _tree(jnp.maximum, vs))
        return mn, mx
    return result


@functools.partial(
    pl.kernel,
    mesh=_mesh,
    out_type=jax.ShapeDtypeStruct((_ROWS, _COLS), jnp.float32),
    scratch_types=[
        pltpu.VMEM((_HALF, _COLS), jnp.float32),
        pltpu.VMEM((_SB, _COLS), jnp.float32),
        pltpu.VMEM((_SB, _COLS), jnp.float32),
        pltpu.VMEM((2, _L), jnp.float32),
        pltpu.VMEM((2 * _NS, _L), jnp.float32),
        pltpu.VMEM_SHARED((2 * _NS, _L), jnp.float32),
        pltpu.SemaphoreType.DMA,
        pltpu.SemaphoreType.DMA,
        pltpu.SemaphoreType.DMA,
        pltpu.SemaphoreType.DMA,
    ],
    compiler_params=_cparams,
)
def _sc_kernel(x_hbm, out_hbm, data, sb0, sb1, mm_loc, gath, shared,
               sem_a, sem_b0, sem_b1, sem_o):
    cid = lax.axis_index("c")
    sid = lax.axis_index("s")
    row0 = sid * _RPT
    mine0 = row0 + cid * _HALF
    oth0 = row0 + (1 - cid) * _HALF

    cp_mine = pltpu.async_copy(x_hbm.at[pl.ds(mine0, _HALF)], data, sem_a)
    sbufs = (sb0, sb1)
    ssems = (sem_b0, sem_b1)
    stream = [pltpu.async_copy(x_hbm.at[pl.ds(oth0, _SB)], sb0, sem_b0)]

    inf = jnp.full((_L,), jnp.inf, jnp.float32)
    cp_mine.wait()
    mn, mx = _reduce_rows(data, _HALF, inf, -inf)

    for blk in range(_NSB):
        if blk + 1 < _NSB:
            stream.append(pltpu.async_copy(
                x_hbm.at[pl.ds(oth0 + (blk + 1) * _SB, _SB)],
                sbufs[(blk + 1) % 2], ssems[(blk + 1) % 2]))
        stream[blk].wait()
        mn, mx = _reduce_rows(sbufs[blk % 2], _SB, mn, mx)

    mm_loc.at[0][...] = mn
    mm_loc.at[1][...] = mx
    pltpu.sync_copy(mm_loc, shared.at[pl.ds(2 * sid, 2)])
    plsc.subcore_barrier()
    pltpu.sync_copy(shared, gath)
    xmin = jnp.min(_tree(jnp.minimum, [gath[2 * w] for w in range(_NS)]))
    xmax = jnp.max(_tree(jnp.maximum, [gath[2 * w + 1] for w in range(_NS)]))

    rng = xmax - xmin
    step = rng * (1.0 / _LEVELS)
    inv_step = _LEVELS * _recip(jnp.where(rng > 0, rng, 1.0))
    base = xmin + step * 0.5

    copies = []
    for chunk in range(_HALF // _CH):
        lo = chunk * _CH

        @pl.loop(0, _CH, unroll=2)
        def _(r):
            row = lo + r
            for c in range(_SLC):
                v = data.at[row, pl.ds(c * _L, _L)][...]
                idx = ((v - xmin) * inv_step).astype(jnp.int32)
                idx = jnp.minimum(idx, _LEVELS - 1)
                q = base + idx.astype(jnp.float32) * step
                data.at[row, pl.ds(c * _L, _L)][...] = q

        copies.append(pltpu.async_copy(
            data.at[pl.ds(lo, _CH)],
            out_hbm.at[pl.ds(mine0 + lo, _CH)], sem_o))
    for cp in copies:
        cp.wait()


def kernel(x):
    b, ch, h, w = x.shape
    xv = x.transpose(0, 2, 3, 1).reshape(_ROWS, _COLS)
    out = _sc_kernel(xv)
    return out.reshape(b, h, w, ch).transpose(0, 3, 1, 2)
